# baseline (speedup 1.0000x reference)
import jax
import jax.numpy as jnp
from jax import lax
from jax.experimental import pallas as pl
from jax.experimental.pallas import tpu as pltpu
from jax.experimental.pallas import tpu_sc as plsc

_VOCAB = 100000
_EMBED = 128
_B = 4096
_L = 50

_NC = 2
_NS = 16
_NW = _NC * _NS

_SEGS = 3 * _B
_SEGS_PER_W = _SEGS // _NW
_SEG_PER_CHUNK = 2
_IDX_PER_CHUNK = _SEG_PER_CHUNK * _L
_CHUNKS_PER_W = _SEGS_PER_W // _SEG_PER_CHUNK
_IDX_ROWS = _SEGS // _SEG_PER_CHUNK
_LANES = 16
_COLS = _EMBED // _LANES


_NBUF = 6
_ROW_UNROLL = 6


_CHUNKS_PER_T = _CHUNKS_PER_W // 3
_ROWS_PER_T = _CHUNKS_PER_T * _SEG_PER_CHUNK


def _sc_body(q_hbm, a_hbm, an_hbm, table_hbm,
             oq_hbm, oa_hbm, oq2_hbm, oan_hbm,
             idx_v, rows_bufs, ostage, sems, osems):
    wid = lax.axis_index("s") * _NC + lax.axis_index("c")
    obase = wid * _ROWS_PER_T

    idx_copies = [
        pltpu.make_async_copy(
            src.at[pl.ds(wid * _CHUNKS_PER_T, _CHUNKS_PER_T)],
            idx_v.at[pl.ds(t * _CHUNKS_PER_T, _CHUNKS_PER_T)],
            osems[0])
        for t, src in enumerate((q_hbm, a_hbm, an_hbm))]
    for cp in idx_copies:
        cp.start()
    for cp in idx_copies:
        cp.wait()

    def start_gather(c_local, rows_v, sem):
        pltpu.make_async_copy(
            table_hbm.at[idx_v.at[c_local]], rows_v, sem).start()

    def wait_gather(rows_v, sem):
        pltpu.make_async_copy(
            table_hbm.at[idx_v.at[0]], rows_v, sem).wait()

    def wait_ostore(j):
        pltpu.make_async_copy(
            ostage[j], oq_hbm.at[pl.ds(0, _SEG_PER_CHUNK)], osems[j]).wait()

    def reduce_chunk(rows_v, j):
        for s in range(_SEG_PER_CHUNK):
            base = s * _L
            accs = tuple(rows_v[base, pl.ds(col * _LANES, _LANES)]
                         for col in range(_COLS))
            def row_body(k, accs):
                r0 = base + 1 + k * _ROW_UNROLL
                for u in range(_ROW_UNROLL):
                    accs = tuple(
                        accs[col] + rows_v[r0 + u, pl.ds(col * _LANES, _LANES)]
                        for col in range(_COLS))
                return accs

            accs = lax.fori_loop(0, 48 // _ROW_UNROLL, row_body, accs)
            for col in range(_COLS):
                ostage[j][s, pl.ds(col * _LANES, _LANES)] = (
                    accs[col] + rows_v[base + _L - 1,
                                       pl.ds(col * _LANES, _LANES)])

    for j in range(_NBUF):
        start_gather(j, rows_bufs[j], sems[j])

    def loop_body(k, carry):
        for j in range(_NBUF):
            c = _NBUF * k + j
            wait_gather(rows_bufs[j], sems[j])

            @pl.when(c >= _NBUF)
            def _():
                wait_ostore(j)

            @pl.when(jnp.logical_and(c >= _NBUF,
                                     c - _NBUF < _CHUNKS_PER_T))
            def _():
                wait_ostore(j)

            reduce_chunk(rows_bufs[j], j)

            orow = obase + (c % _CHUNKS_PER_T) * _SEG_PER_CHUNK

            @pl.when(c < _CHUNKS_PER_T)
            def _():
                pltpu.make_async_copy(
                    ostage[j], oq_hbm.at[pl.ds(orow, _SEG_PER_CHUNK)],
                    osems[j]).start()
                pltpu.make_async_copy(
                    ostage[j], oq2_hbm.at[pl.ds(orow, _SEG_PER_CHUNK)],
                    osems[j]).start()

            @pl.when(jnp.logical_and(c >= _CHUNKS_PER_T,
                                     c < 2 * _CHUNKS_PER_T))
            def _():
                pltpu.make_async_copy(
                    ostage[j], oa_hbm.at[pl.ds(orow, _SEG_PER_CHUNK)],
                    osems[j]).start()

            @pl.when(c >= 2 * _CHUNKS_PER_T)
            def _():
                pltpu.make_async_copy(
                    ostage[j], oan_hbm.at[pl.ds(orow, _SEG_PER_CHUNK)],
                    osems[j]).start()

            @pl.when(c + _NBUF < _CHUNKS_PER_W)
            def _():
                start_gather(c + _NBUF, rows_bufs[j], sems[j])

        return carry

    lax.fori_loop(0, _CHUNKS_PER_W // _NBUF, loop_body, 0)

    for j in range(_NBUF):
        wait_ostore(j)


@jax.jit
def _unigram_pooled(q2, a2, an2, embedding):
    mesh = plsc.VectorSubcoreMesh(core_axis_name="c", subcore_axis_name="s")
    enc = jax.ShapeDtypeStruct((_B, _EMBED), jnp.float32)
    kern = pl.kernel(
        _sc_body,
        out_type=(enc, enc, enc, enc),
        mesh=mesh,
        scratch_types=[
            pltpu.VMEM((_CHUNKS_PER_W, _IDX_PER_CHUNK), jnp.int32),
            [pltpu.VMEM((_IDX_PER_CHUNK, _EMBED), jnp.float32)
             for _ in range(_NBUF)],
            [pltpu.VMEM((_SEG_PER_CHUNK, _EMBED), jnp.float32)
             for _ in range(_NBUF)],
            [pltpu.SemaphoreType.DMA for _ in range(_NBUF)],
            [pltpu.SemaphoreType.DMA for _ in range(_NBUF)],
        ],
    )
    return kern(q2, a2, an2, embedding)


def kernel(q, a, a_neg, embedding):
    shape2 = (_B * _L // _IDX_PER_CHUNK, _IDX_PER_CHUNK)
    oq, oa, oq2, oan = _unigram_pooled(
        q.reshape(shape2), a.reshape(shape2), a_neg.reshape(shape2),
        embedding)
    return (oq, oa, oq2, oan)


# speedup vs baseline: 1.0579x; 1.0579x over previous
import jax
import jax.numpy as jnp
from jax import lax
from jax.experimental import pallas as pl
from jax.experimental.pallas import tpu as pltpu
from jax.experimental.pallas import tpu_sc as plsc

_VOCAB = 100000
_EMBED = 128
_B = 4096
_L = 50

_NC = 2
_NS = 16
_NW = _NC * _NS

_SEGS = 3 * _B
_SEGS_PER_W = _SEGS // _NW
_SEG_PER_CHUNK = 2
_IDX_PER_CHUNK = _SEG_PER_CHUNK * _L
_CHUNKS_PER_W = _SEGS_PER_W // _SEG_PER_CHUNK
_IDX_ROWS = _SEGS // _SEG_PER_CHUNK
_LANES = 16
_COLS = _EMBED // _LANES


_NBUF = 6
_ROW_UNROLL = 6


_CHUNKS_PER_T = _CHUNKS_PER_W // 3
_ROWS_PER_T = _CHUNKS_PER_T * _SEG_PER_CHUNK


def _sc_body(q_hbm, a_hbm, an_hbm, table_hbm,
             oq_hbm, oa_hbm, oq2_hbm, oan_hbm,
             idx_v, rows_bufs, ostage, sems, osems):
    wid = lax.axis_index("s") * _NC + lax.axis_index("c")
    obase = wid * _ROWS_PER_T

    for t, src in enumerate((q_hbm, a_hbm, an_hbm)):
        pltpu.sync_copy(src.at[pl.ds(wid * _CHUNKS_PER_T, _CHUNKS_PER_T)],
                        idx_v.at[pl.ds(t * _CHUNKS_PER_T, _CHUNKS_PER_T)])

    def start_gather(c_local, rows_v, sem):
        pltpu.make_async_copy(
            table_hbm.at[idx_v.at[c_local]], rows_v, sem).start()

    def wait_gather(rows_v, sem):
        pltpu.make_async_copy(
            table_hbm.at[idx_v.at[0]], rows_v, sem).wait()

    def wait_ostore(j):
        pltpu.make_async_copy(
            ostage[j], oq_hbm.at[pl.ds(0, _SEG_PER_CHUNK)], osems[j]).wait()

    def reduce_chunk(rows_v, j):
        for s in range(_SEG_PER_CHUNK):
            base = s * _L
            accs = tuple(rows_v[base, pl.ds(col * _LANES, _LANES)]
                         for col in range(_COLS))
            def row_body(k, accs):
                r0 = base + 1 + k * _ROW_UNROLL
                for u in range(_ROW_UNROLL):
                    accs = tuple(
                        accs[col] + rows_v[r0 + u, pl.ds(col * _LANES, _LANES)]
                        for col in range(_COLS))
                return accs

            accs = lax.fori_loop(0, 48 // _ROW_UNROLL, row_body, accs)
            for col in range(_COLS):
                ostage[j][s, pl.ds(col * _LANES, _LANES)] = (
                    accs[col] + rows_v[base + _L - 1,
                                       pl.ds(col * _LANES, _LANES)])

    for j in range(_NBUF):
        start_gather(j, rows_bufs[j], sems[j])

    def loop_body(k, carry):
        for j in range(_NBUF):
            c = _NBUF * k + j
            wait_gather(rows_bufs[j], sems[j])

            @pl.when(c >= _NBUF)
            def _():
                wait_ostore(j)

            @pl.when(jnp.logical_and(c >= _NBUF,
                                     c - _NBUF < _CHUNKS_PER_T))
            def _():
                wait_ostore(j)

            reduce_chunk(rows_bufs[j], j)

            orow = obase + (c % _CHUNKS_PER_T) * _SEG_PER_CHUNK

            @pl.when(c < _CHUNKS_PER_T)
            def _():
                pltpu.make_async_copy(
                    ostage[j], oq_hbm.at[pl.ds(orow, _SEG_PER_CHUNK)],
                    osems[j]).start()
                pltpu.make_async_copy(
                    ostage[j], oq2_hbm.at[pl.ds(orow, _SEG_PER_CHUNK)],
                    osems[j]).start()

            @pl.when(jnp.logical_and(c >= _CHUNKS_PER_T,
                                     c < 2 * _CHUNKS_PER_T))
            def _():
                pltpu.make_async_copy(
                    ostage[j], oa_hbm.at[pl.ds(orow, _SEG_PER_CHUNK)],
                    osems[j]).start()

            @pl.when(c >= 2 * _CHUNKS_PER_T)
            def _():
                pltpu.make_async_copy(
                    ostage[j], oan_hbm.at[pl.ds(orow, _SEG_PER_CHUNK)],
                    osems[j]).start()

            @pl.when(c + _NBUF < _CHUNKS_PER_W)
            def _():
                start_gather(c + _NBUF, rows_bufs[j], sems[j])

        return carry

    lax.fori_loop(0, _CHUNKS_PER_W // _NBUF, loop_body, 0)

    for j in range(_NBUF):
        wait_ostore(j)


@jax.jit
def _unigram_pooled(q2, a2, an2, embedding):
    mesh = plsc.VectorSubcoreMesh(core_axis_name="c", subcore_axis_name="s")
    enc = jax.ShapeDtypeStruct((_B, _EMBED), jnp.float32)
    kern = pl.kernel(
        _sc_body,
        out_type=(enc, enc, enc, enc),
        mesh=mesh,
        scratch_types=[
            pltpu.VMEM((_CHUNKS_PER_W, _IDX_PER_CHUNK), jnp.int32),
            [pltpu.VMEM((_IDX_PER_CHUNK, _EMBED), jnp.float32)
             for _ in range(_NBUF)],
            [pltpu.VMEM((_SEG_PER_CHUNK, _EMBED), jnp.float32)
             for _ in range(_NBUF)],
            [pltpu.SemaphoreType.DMA for _ in range(_NBUF)],
            [pltpu.SemaphoreType.DMA for _ in range(_NBUF)],
        ],
    )
    return kern(q2, a2, an2, embedding)


def kernel(q, a, a_neg, embedding):
    shape2 = (_B * _L // _IDX_PER_CHUNK, _IDX_PER_CHUNK)
    oq, oa, oq2, oan = _unigram_pooled(
        q.reshape(shape2), a.reshape(shape2), a_neg.reshape(shape2),
        embedding)
    return (oq, oa, oq2, oan)
